# baseline (speedup 1.0000x reference)
import functools

import jax
import jax.numpy as jnp
from jax import lax
from jax.experimental import pallas as pl
from jax.experimental.pallas import tpu as pltpu
from jax.experimental.pallas import tpu_sc as plsc

LANES = 16
NC, NS = 2, 16
NW = NC * NS
CHUNK = 128


def _rsqrt(v):
    bits = lax.bitcast_convert_type(v, jnp.int32)
    y = lax.bitcast_convert_type(jnp.int32(0x5F3759DF) - (bits >> 1), jnp.float32)
    for _ in range(3):
        y = y * (jnp.float32(1.5) - jnp.float32(0.5) * v * y * y)
    return y


def _make_sc_kernel(n_tokens, vocab, emb, n_pos):
    per_w = n_tokens // NW
    n_chunks = per_w // CHUNK
    n_groups = CHUNK // LANES
    nsub = emb // LANES
    mesh = plsc.VectorSubcoreMesh(core_axis_name="c", subcore_axis_name="s")

    @functools.partial(
        pl.kernel,
        out_type=jax.ShapeDtypeStruct((n_tokens, emb), jnp.float32),
        mesh=mesh,
        compiler_params=pltpu.CompilerParams(
            use_tc_tiling_on_sc=False, needs_layout_passes=False),
        scratch_types=[
            pltpu.VMEM((CHUNK,), jnp.int32),
            pltpu.VMEM((CHUNK,), jnp.int32),
            pltpu.VMEM((CHUNK,), jnp.int32),
            pltpu.VMEM((CHUNK, emb), jnp.float32),
            pltpu.VMEM((CHUNK, emb), jnp.float32),
            pltpu.VMEM((n_pos, emb), jnp.float32),
            pltpu.VMEM((emb,), jnp.float32),
            pltpu.VMEM((emb,), jnp.float32),
            pltpu.SemaphoreType.DMA,
        ],
    )
    def sc_kernel(src_h, seg_h, word_h, pos_h, g_h, b_h, out_h,
                  idx_v, seg_v, pidx_v, wbuf, obuf, posv, gv, bv, sem):
        wid = lax.axis_index("s") * NC + lax.axis_index("c")
        pltpu.sync_copy(pos_h, posv)
        pltpu.sync_copy(g_h, gv)
        pltpu.sync_copy(b_h, bv)
        base0 = wid * per_w
        gs = [gv[pl.ds(LANES * k, LANES)] for k in range(nsub)]
        bs = [bv[pl.ds(LANES * k, LANES)] for k in range(nsub)]
        inv_e = jnp.float32(1.0 / emb)
        iota = lax.iota(jnp.int32, LANES)

        def chunk_body(ci, _):
            base = base0 + ci * CHUNK
            pltpu.sync_copy(src_h.at[pl.ds(base, CHUNK)], idx_v)
            pltpu.sync_copy(seg_h.at[pl.ds(base, CHUNK)], seg_v)

            def pix_body(j, _):
                s = seg_v[pl.ds(j * LANES, LANES)]
                q = (s.astype(jnp.float32) * jnp.float32(1.0 / 10000.0)).astype(jnp.int32)
                r = s - q * 10000
                q = jnp.where(r >= 10000, q + 1, q)
                q = jnp.where(r < 0, q - 1, q)
                pidx_v[pl.ds(j * LANES, LANES)] = q
                return 0

            lax.fori_loop(0, n_groups, pix_body, 0)
            pltpu.async_copy(word_h.at[idx_v], wbuf, sem).wait()

            def group_body(g, _):
                rowv = iota + g * LANES
                pvec = pidx_v[pl.ds(g * LANES, LANES)]
                acc = [jnp.zeros((LANES,), jnp.float32) for _ in range(4)]
                acc2 = [jnp.zeros((LANES,), jnp.float32) for _ in range(4)]
                for d in range(emb):
                    dsplat = jnp.full((LANES,), d, jnp.int32)
                    x = (plsc.load_gather(wbuf, [rowv, dsplat])
                         + plsc.load_gather(posv, [pvec, dsplat]))
                    acc[d % 4] = acc[d % 4] + x
                    acc2[d % 4] = acc2[d % 4] + x * x
                    plsc.store_scatter(obuf, [rowv, dsplat], x)
                sumv = (acc[0] + acc[1]) + (acc[2] + acc[3])
                sumsqv = (acc2[0] + acc2[1]) + (acc2[2] + acc2[3])
                meanvec = sumv * inv_e
                varvec = sumsqv * inv_e - meanvec * meanvec
                rstdvec = _rsqrt(varvec + jnp.float32(1e-6))
                for i in range(LANES):
                    t = g * LANES + i
                    msp = jnp.full((LANES,), meanvec[i], jnp.float32)
                    rsp = jnp.full((LANES,), rstdvec[i], jnp.float32)
                    for k in range(nsub):
                        xk = obuf[t, pl.ds(LANES * k, LANES)]
                        obuf[t, pl.ds(LANES * k, LANES)] = (
                            (xk - msp) * rsp * gs[k] + bs[k])
                return 0

            lax.fori_loop(0, n_groups, group_body, 0)
            pltpu.sync_copy(obuf, out_h.at[pl.ds(base, CHUNK)])
            return 0

        lax.fori_loop(0, n_chunks, chunk_body, 0)

    return sc_kernel


def kernel(src, seg, word_table, pos_table, gamma, beta):
    b, l = src.shape
    vocab, emb = word_table.shape
    n_pos = pos_table.shape[0]
    n = b * l
    flat_src = src.reshape(n).astype(jnp.int32)
    flat_seg = seg.reshape(n).astype(jnp.int32)
    sc = _make_sc_kernel(n, vocab, emb, n_pos)
    out = sc(flat_src, flat_seg, word_table, pos_table, gamma, beta)
    return out.reshape(b, l, emb)


# speedup vs baseline: 1.2776x; 1.2776x over previous
import functools

import jax
import jax.numpy as jnp
from jax import lax
from jax.experimental import pallas as pl
from jax.experimental.pallas import tpu as pltpu
from jax.experimental.pallas import tpu_sc as plsc

LANES = 16
NC, NS = 2, 16
NW = NC * NS
CHUNK = 128


def _rsqrt(v):
    bits = lax.bitcast_convert_type(v, jnp.int32)
    y = lax.bitcast_convert_type(jnp.int32(0x5F3759DF) - (bits >> 1), jnp.float32)
    for _ in range(3):
        y = y * (jnp.float32(1.5) - jnp.float32(0.5) * v * y * y)
    return y


def _make_sc_kernel(n_tokens, vocab, emb, n_pos):
    per_w = n_tokens // NW
    n_chunks = per_w // CHUNK
    n_groups = CHUNK // LANES
    nsub = emb // LANES
    assert n_chunks % 2 == 0
    mesh = plsc.VectorSubcoreMesh(core_axis_name="c", subcore_axis_name="s")

    @functools.partial(
        pl.kernel,
        out_type=jax.ShapeDtypeStruct((n_tokens, emb), jnp.float32),
        mesh=mesh,
        compiler_params=pltpu.CompilerParams(
            use_tc_tiling_on_sc=False, needs_layout_passes=False),
        scratch_types=[
            pltpu.VMEM((CHUNK,), jnp.int32),
            pltpu.VMEM((CHUNK,), jnp.int32),
            pltpu.VMEM((CHUNK,), jnp.int32),
            pltpu.VMEM((CHUNK,), jnp.int32),
            pltpu.VMEM((CHUNK,), jnp.int32),
            pltpu.VMEM((CHUNK, emb), jnp.float32),
            pltpu.VMEM((CHUNK, emb), jnp.float32),
            pltpu.VMEM((CHUNK, emb), jnp.float32),
            pltpu.VMEM((CHUNK, emb), jnp.float32),
            pltpu.VMEM((n_pos, emb), jnp.float32),
            pltpu.VMEM((emb,), jnp.float32),
            pltpu.VMEM((emb,), jnp.float32),
            pltpu.SemaphoreType.DMA,
            pltpu.SemaphoreType.DMA,
            pltpu.SemaphoreType.DMA,
            pltpu.SemaphoreType.DMA,
        ],
    )
    def sc_kernel(src_h, seg_h, word_h, pos_h, g_h, b_h, out_h,
                  idx0, idx1, seg_v, pidx0, pidx1, wbuf0, wbuf1,
                  obuf0, obuf1, posv, gv, bv, gsem0, gsem1, ssem0, ssem1):
        wid = lax.axis_index("s") * NC + lax.axis_index("c")
        pltpu.sync_copy(pos_h, posv)
        pltpu.sync_copy(g_h, gv)
        pltpu.sync_copy(b_h, bv)
        base0 = wid * per_w
        gs = [gv[pl.ds(LANES * k, LANES)] for k in range(nsub)]
        bs = [bv[pl.ds(LANES * k, LANES)] for k in range(nsub)]
        inv_e = jnp.float32(1.0 / emb)
        iota = lax.iota(jnp.int32, LANES)
        ring = ((idx0, pidx0, wbuf0, gsem0), (idx1, pidx1, wbuf1, gsem1))
        oring = ((obuf0, ssem0), (obuf1, ssem1))

        def stage(ci, idx_r, pidx_r):
            base = base0 + ci * CHUNK
            pltpu.sync_copy(src_h.at[pl.ds(base, CHUNK)], idx_r)
            pltpu.sync_copy(seg_h.at[pl.ds(base, CHUNK)], seg_v)

            def pix_body(j, _):
                s = seg_v[pl.ds(j * LANES, LANES)]
                q = (s.astype(jnp.float32) * jnp.float32(1.0 / 10000.0)).astype(jnp.int32)
                r = s - q * 10000
                q = jnp.where(r >= 10000, q + 1, q)
                q = jnp.where(r < 0, q - 1, q)
                pidx_r[pl.ds(j * LANES, LANES)] = q
                return 0

            lax.fori_loop(0, n_groups, pix_body, 0)

        def compute(wbuf, pidx_r, obuf):
            def group_body(g):
                rowv = iota + g * LANES
                pvec = pidx_r[pl.ds(g * LANES, LANES)]
                acc = [jnp.zeros((LANES,), jnp.float32) for _ in range(4)]
                acc2 = [jnp.zeros((LANES,), jnp.float32) for _ in range(4)]
                for d in range(emb):
                    dsplat = jnp.full((LANES,), d, jnp.int32)
                    x = (plsc.load_gather(wbuf, [rowv, dsplat])
                         + plsc.load_gather(posv, [pvec, dsplat]))
                    acc[d % 4] = acc[d % 4] + x
                    acc2[d % 4] = acc2[d % 4] + x * x
                    plsc.store_scatter(obuf, [rowv, dsplat], x)
                sumv = (acc[0] + acc[1]) + (acc[2] + acc[3])
                sumsqv = (acc2[0] + acc2[1]) + (acc2[2] + acc2[3])
                meanvec = sumv * inv_e
                varvec = sumsqv * inv_e - meanvec * meanvec
                rstdvec = _rsqrt(varvec + jnp.float32(1e-6))
                for i in range(LANES):
                    t = g * LANES + i
                    msp = jnp.full((LANES,), meanvec[i], jnp.float32)
                    rsp = jnp.full((LANES,), rstdvec[i], jnp.float32)
                    for k in range(nsub):
                        xk = obuf[t, pl.ds(LANES * k, LANES)]
                        obuf[t, pl.ds(LANES * k, LANES)] = (
                            (xk - msp) * rsp * gs[k] + bs[k])

            plsc.parallel_loop(0, n_groups, 1, unroll=1)(group_body)

        stage(0, idx0, pidx0)
        pltpu.make_async_copy(word_h.at[idx0], wbuf0, gsem0).start()

        def outer_body(cg, _):
            for b in range(2):
                ci = cg * 2 + b
                idx_c, pidx_c, wbuf_c, gsem_c = ring[b]
                idx_n, pidx_n, wbuf_n, gsem_n = ring[(b + 1) % 2]
                obuf_c, ssem_c = oring[b]
                nci = ci + 1

                @pl.when(nci < n_chunks)
                def _():
                    stage(nci, idx_n, pidx_n)
                    pltpu.make_async_copy(
                        word_h.at[idx_n], wbuf_n, gsem_n).start()

                pltpu.make_async_copy(word_h.at[idx_c], wbuf_c, gsem_c).wait()

                @pl.when(ci >= 2)
                def _():
                    pbase = base0 + (ci - 2) * CHUNK
                    pltpu.make_async_copy(
                        obuf_c, out_h.at[pl.ds(pbase, CHUNK)], ssem_c).wait()

                compute(wbuf_c, pidx_c, obuf_c)
                base = base0 + ci * CHUNK
                pltpu.make_async_copy(
                    obuf_c, out_h.at[pl.ds(base, CHUNK)], ssem_c).start()
            return 0

        lax.fori_loop(0, n_chunks // 2, outer_body, 0)
        for b, ci in ((0, n_chunks - 2), (1, n_chunks - 1)):
            obuf_c, ssem_c = oring[b]
            base = base0 + ci * CHUNK
            pltpu.make_async_copy(
                obuf_c, out_h.at[pl.ds(base, CHUNK)], ssem_c).wait()

    return sc_kernel


def kernel(src, seg, word_table, pos_table, gamma, beta):
    b, l = src.shape
    vocab, emb = word_table.shape
    n_pos = pos_table.shape[0]
    n = b * l
    flat_src = src.reshape(n).astype(jnp.int32)
    flat_seg = seg.reshape(n).astype(jnp.int32)
    sc = _make_sc_kernel(n, vocab, emb, n_pos)
    out = sc(flat_src, flat_seg, word_table, pos_table, gamma, beta)
    return out.reshape(b, l, emb)


# speedup vs baseline: 2.1947x; 1.7179x over previous
import functools

import jax
import jax.numpy as jnp
from jax import lax
from jax.experimental import pallas as pl
from jax.experimental.pallas import tpu as pltpu
from jax.experimental.pallas import tpu_sc as plsc

LANES = 16
NC, NS = 2, 16
NW = NC * NS
CHUNK = 128
WAVE = 4 * CHUNK
HALF = WAVE // 2


def _rsqrt(v):
    bits = lax.bitcast_convert_type(v, jnp.int32)
    y = lax.bitcast_convert_type(jnp.int32(0x5F3759DF) - (bits >> 1), jnp.float32)
    for _ in range(3):
        y = y * (jnp.float32(1.5) - jnp.float32(0.5) * v * y * y)
    return y


def _make_sc_kernel(n_tokens, vocab, emb, n_pos):
    per_w = n_tokens // NW
    n_waves = per_w // WAVE
    groups_per_half = HALF // LANES
    nsub = emb // LANES
    assert per_w % WAVE == 0
    mesh = plsc.VectorSubcoreMesh(core_axis_name="c", subcore_axis_name="s")

    @functools.partial(
        pl.kernel,
        out_type=jax.ShapeDtypeStruct((n_tokens * emb,), jnp.float32),
        mesh=mesh,
        compiler_params=pltpu.CompilerParams(
            use_tc_tiling_on_sc=False, needs_layout_passes=False),
        scratch_types=[
            pltpu.VMEM((WAVE,), jnp.int32),
            pltpu.VMEM((WAVE,), jnp.int32),
            pltpu.VMEM((WAVE,), jnp.int32),
            pltpu.VMEM((WAVE,), jnp.int32),
            pltpu.VMEM((WAVE, emb), jnp.float32),
            pltpu.VMEM((HALF * emb,), jnp.float32),
            pltpu.VMEM((HALF * emb,), jnp.float32),
            pltpu.VMEM((n_pos, emb), jnp.float32),
            pltpu.VMEM((emb,), jnp.float32),
            pltpu.VMEM((emb,), jnp.float32),
            pltpu.SemaphoreType.DMA,
            pltpu.SemaphoreType.DMA,
            pltpu.SemaphoreType.DMA,
            pltpu.SemaphoreType.DMA,
            pltpu.SemaphoreType.DMA,
            pltpu.SemaphoreType.DMA,
        ],
    )
    def sc_kernel(src_h, seg_h, word_h, pos_h, g_h, b_h, out_h,
                  idxn, segb, pidxn, pidxc, wbuf, obuf0, obuf1, posv, gv, bv,
                  g0, g1, g2, g3, s0, s1):
        wid = lax.axis_index("s") * NC + lax.axis_index("c")
        pltpu.sync_copy(pos_h, posv)
        pltpu.sync_copy(g_h, gv)
        pltpu.sync_copy(b_h, bv)
        base0 = wid * per_w
        gs = [gv[pl.ds(LANES * k, LANES)] for k in range(nsub)]
        bs = [bv[pl.ds(LANES * k, LANES)] for k in range(nsub)]
        inv_e = jnp.float32(1.0 / emb)
        iota = lax.iota(jnp.int32, LANES)
        gsems = (g0, g1, g2, g3)
        obufs = (obuf0, obuf1)
        ssems = (s0, s1)

        def gather_q(b):
            return pltpu.make_async_copy(
                word_h.at[idxn.at[pl.ds(b * CHUNK, CHUNK)]],
                wbuf.at[pl.ds(b * CHUNK, CHUNK)],
                gsems[b])

        def store_h(h, w):
            base = (base0 + w * WAVE + h * HALF) * emb
            return pltpu.make_async_copy(
                obufs[h], out_h.at[pl.ds(base, HALF * emb)], ssems[h])

        def stage(w):
            base = base0 + w * WAVE
            pltpu.sync_copy(src_h.at[pl.ds(base, WAVE)], idxn)
            pltpu.sync_copy(seg_h.at[pl.ds(base, WAVE)], segb)

            def pix_body(j):
                s = segb[pl.ds(j * LANES, LANES)]
                q = (s.astype(jnp.float32) * jnp.float32(1.0 / 10000.0)).astype(jnp.int32)
                r = s - q * 10000
                q = jnp.where(r >= 10000, q + 1, q)
                q = jnp.where(r < 0, q - 1, q)
                pidxn[pl.ds(j * LANES, LANES)] = q

            plsc.parallel_loop(0, WAVE // LANES, 1)(pix_body)

        def compute_half(h):
            obuf = obufs[h]

            def group_body(g):
                lrow = g * LANES
                wrow = h * HALF + lrow
                rowv = iota + wrow
                oflat = (iota + lrow) * emb
                pvec = pidxc[pl.ds(wrow, LANES)]
                acc = [jnp.zeros((LANES,), jnp.float32) for _ in range(4)]
                acc2 = [jnp.zeros((LANES,), jnp.float32) for _ in range(4)]
                for d in range(emb):
                    dskew = (iota + d) & (emb - 1)
                    x = (plsc.load_gather(wbuf, [rowv, dskew])
                         + plsc.load_gather(posv, [pvec, dskew]))
                    acc[d % 4] = acc[d % 4] + x
                    acc2[d % 4] = acc2[d % 4] + x * x
                    plsc.store_scatter(obuf, [oflat + dskew], x)
                sumv = (acc[0] + acc[1]) + (acc[2] + acc[3])
                sumsqv = (acc2[0] + acc2[1]) + (acc2[2] + acc2[3])
                meanvec = sumv * inv_e
                varvec = sumsqv * inv_e - meanvec * meanvec
                rstdvec = _rsqrt(varvec + jnp.float32(1e-6))
                for i in range(LANES):
                    t = lrow + i
                    msp = jnp.full((LANES,), meanvec[i], jnp.float32)
                    rsp = jnp.full((LANES,), rstdvec[i], jnp.float32)
                    for k in range(nsub):
                        sl = pl.ds(t * emb + LANES * k, LANES)
                        xk = obuf[sl]
                        obuf[sl] = (xk - msp) * rsp * gs[k] + bs[k]

            plsc.parallel_loop(0, groups_per_half, 1)(group_body)

        def copy_pidx():
            def cp(j):
                pidxc[pl.ds(j * LANES, LANES)] = pidxn[pl.ds(j * LANES, LANES)]
            plsc.parallel_loop(0, WAVE // LANES, 1)(cp)

        stage(0)
        for b in range(4):
            gather_q(b).start()

        def wave_body(w, _):
            copy_pidx()

            gather_q(0).wait()
            gather_q(1).wait()

            @pl.when(w >= 1)
            def _():
                store_h(0, w - 1).wait()

            compute_half(0)
            store_h(0, w).start()

            gather_q(2).wait()
            gather_q(3).wait()

            @pl.when(w < n_waves - 1)
            def _():
                stage(w + 1)
                gather_q(0).start()
                gather_q(1).start()

            @pl.when(w >= 1)
            def _():
                store_h(1, w - 1).wait()

            compute_half(1)
            store_h(1, w).start()

            @pl.when(w < n_waves - 1)
            def _():
                gather_q(2).start()
                gather_q(3).start()

            return 0

        lax.fori_loop(0, n_waves, wave_body, 0)
        store_h(0, n_waves - 1).wait()
        store_h(1, n_waves - 1).wait()

    return sc_kernel


def kernel(src, seg, word_table, pos_table, gamma, beta):
    b, l = src.shape
    vocab, emb = word_table.shape
    n_pos = pos_table.shape[0]
    n = b * l
    flat_src = src.reshape(n).astype(jnp.int32)
    flat_seg = seg.reshape(n).astype(jnp.int32)
    sc = _make_sc_kernel(n, vocab, emb, n_pos)
    out = sc(flat_src, flat_seg, word_table, pos_table, gamma, beta)
    return out.reshape(b, l, emb)
